# baseline (speedup 1.0000x reference)
import functools

import jax
import jax.numpy as jnp
from jax import lax
from jax.experimental import pallas as pl
from jax.experimental.pallas import tpu as pltpu
from jax.experimental.pallas import tpu_sc as plsc

N_VOCAB = 1000000
N_EMBED = 64
BATCH = 16384

NC = 2
NS = 16
NW = NC * NS
B_PER_W = BATCH // NW
CHUNK = 128
N_CHUNKS = B_PER_W // CHUNK


def _gather_body(idx_hbm, table_hbm, out_hbm, idx_v, rows_v, sem):
    wid = lax.axis_index("s") * NC + lax.axis_index("c")
    base = wid * B_PER_W
    pltpu.sync_copy(idx_hbm.at[wid], idx_v)
    copies = [
        pltpu.async_copy(
            table_hbm.at[idx_v.at[j]],
            rows_v.at[pl.ds(j * CHUNK, CHUNK)],
            sem,
        )
        for j in range(N_CHUNKS)
    ]
    for c in copies:
        c.wait()
    pltpu.sync_copy(rows_v, out_hbm.at[pl.ds(base, B_PER_W)])


@jax.jit
def _gather(idx3, table):
    return pl.kernel(
        _gather_body,
        mesh=plsc.VectorSubcoreMesh(core_axis_name="c", subcore_axis_name="s"),
        out_type=jax.ShapeDtypeStruct((BATCH, N_EMBED), jnp.float32),
        scratch_types=[
            pltpu.VMEM((N_CHUNKS, CHUNK), jnp.int32),
            pltpu.VMEM((B_PER_W, N_EMBED), jnp.float32),
            pltpu.SemaphoreType.DMA,
        ],
        compiler_params=pltpu.CompilerParams(use_tc_tiling_on_sc=False),
    )(idx3, table)


def kernel(input_words, W_in):
    idx3 = input_words.astype(jnp.int32).reshape(NW, N_CHUNKS, CHUNK)
    return _gather(idx3, W_in)


# speedup vs baseline: 2.1343x; 2.1343x over previous
import jax
import jax.numpy as jnp
from jax import lax
from jax.experimental import pallas as pl
from jax.experimental.pallas import tpu as pltpu
from jax.experimental.pallas import tpu_sc as plsc

N_VOCAB = 1000000
N_EMBED = 64
BATCH = 16384

NC = 2
NS = 16
NW = NC * NS
B_PER_W = BATCH // NW
ROWS_PER_TILE = 8
N_TILE_ROWS = N_VOCAB // ROWS_PER_TILE
LANE = 16
F = LANE
N_GROUPS = B_PER_W // F


def _gather_body(*refs):
    idx_hbm, table_hbm, out_hbm = refs[0], refs[1], refs[2]
    idx_v = refs[3]
    bufs = refs[4:4 + F]
    outbuf_v = refs[4 + F]
    sems = refs[5 + F:5 + 2 * F]

    wid = lax.axis_index("s") * NC + lax.axis_index("c")
    base = wid * B_PER_W
    pltpu.sync_copy(idx_hbm.at[pl.ds(base, B_PER_W)], idx_v.at[pl.ds(0, B_PER_W)])

    def fire(hi, b):
        pltpu.async_copy(table_hbm.at[hi], bufs[b], sems[b])

    hv0 = lax.shift_right_logical(idx_v[pl.ds(0, LANE)], 3)
    for b in range(F):
        fire(hv0[b], b)

    def group_body(i, carry):
        g0 = pl.multiple_of(i * F, F)
        rv = lax.bitwise_and(idx_v[pl.ds(g0, LANE)], 7)
        hv_next = lax.shift_right_logical(idx_v[pl.ds(g0 + F, LANE)], 3)
        for b in range(F):
            j = g0 + b
            pltpu.make_async_copy(table_hbm.at[0], bufs[b], sems[b]).wait()
            r = rv[b]
            for c in range(N_EMBED // LANE):
                outbuf_v[j, pl.ds(c * LANE, LANE)] = bufs[b][r, pl.ds(c * LANE, LANE)]
            @pl.when(i < N_GROUPS - 1)
            def _():
                fire(hv_next[b], b)
        return carry

    lax.fori_loop(0, N_GROUPS, group_body, jnp.int32(0))
    pltpu.sync_copy(outbuf_v, out_hbm.at[pl.ds(base, B_PER_W)])


@jax.jit
def _gather(idx, table3):
    return pl.kernel(
        _gather_body,
        mesh=plsc.VectorSubcoreMesh(core_axis_name="c", subcore_axis_name="s"),
        out_type=jax.ShapeDtypeStruct((BATCH, N_EMBED), jnp.float32),
        scratch_types=(
            [pltpu.VMEM((B_PER_W + LANE,), jnp.int32)]
            + [pltpu.VMEM((ROWS_PER_TILE, N_EMBED), jnp.float32) for _ in range(F)]
            + [pltpu.VMEM((B_PER_W, N_EMBED), jnp.float32)]
            + [pltpu.SemaphoreType.DMA for _ in range(F)]
        ),
    )(idx, table3)


def kernel(input_words, W_in):
    idx = input_words.astype(jnp.int32)
    table3 = W_in.reshape(N_TILE_ROWS, ROWS_PER_TILE, N_EMBED)
    return _gather(idx, table3)
